# Initial kernel scaffold; baseline (speedup 1.0000x reference)
import jax
import jax.numpy as jnp
from jax.experimental import pallas as pl


def kernel(x, edge_index, W1l, b1l, W1r, W2l, b2l, W2r):
    raise NotImplementedError("write your pallas kernel here")


# speedup vs baseline: 3.1084x; 3.1084x over previous
import functools

import jax
import jax.numpy as jnp
from jax import lax
from jax.experimental import pallas as pl
from jax.experimental.pallas import tpu as pltpu
from jax.experimental.pallas import tpu_sc as plsc

N = 10000
E = 320000
D = 128
DW = 144

NP = 10240
NW = 32
EP = 327680
EW = EP // NW
C = 128
CHUNKS = EW // C
RPT = NP // 16
DEGW = 16

_MESH = plsc.VectorSubcoreMesh(core_axis_name="c", subcore_axis_name="s")


def _sc_body(with_gather, table, srcs, dsts, zrows, agg_out, src_v, dst_v,
             rows_v, acc, sem):
    c = lax.axis_index("c")
    s = lax.axis_index("s")
    wid = s * 2 + c
    rbase = s * RPT
    obase = c * NP + rbase

    pltpu.sync_copy(zrows, rows_v)
    for jj in range(RPT // C):
        pltpu.sync_copy(rows_v, acc.at[pl.ds(rbase + jj * C, C)])
    if not with_gather:
        pltpu.sync_copy(table.at[pl.ds(0, C)], rows_v)
    plsc.subcore_barrier()

    ebase = wid * EW

    def step(j, carry):
        off = ebase + j * C
        pltpu.sync_copy(dsts.at[pl.ds(off, C)], dst_v)
        if with_gather:
            pltpu.sync_copy(srcs.at[pl.ds(off, C)], src_v)
            pltpu.async_copy(table.at[src_v], rows_v, sem).wait()
        pltpu.sync_copy(rows_v, acc.at[dst_v], add=True)
        return carry

    lax.fori_loop(0, CHUNKS, step, 0)
    plsc.subcore_barrier()

    for jj in range(RPT // C):
        pltpu.sync_copy(acc.at[pl.ds(rbase + jj * C, C)], rows_v)
        pltpu.sync_copy(rows_v, agg_out.at[pl.ds(obase + jj * C, C)])


def _make_sc(with_gather):
    return pl.kernel(
        functools.partial(_sc_body, with_gather),
        out_type=[jax.ShapeDtypeStruct((2 * NP, D), jnp.float32)],
        mesh=_MESH,
        scratch_types=[
            pltpu.VMEM((C,), jnp.int32),
            pltpu.VMEM((C,), jnp.int32),
            pltpu.VMEM((C, D), jnp.float32),
            pltpu.VMEM_SHARED((NP, D), jnp.float32),
            pltpu.SemaphoreType.DMA,
        ],
    )


_sc_agg = _make_sc(True)
_sc_deg = _make_sc(False)

BN = 512
GRID = NP // BN


def _tc_body(final, agg_ref, degs_ref, x_ref, wl_ref, b_ref, wr_ref, o_ref):
    deg = degs_ref[0, :, 0:1] + degs_ref[1, :, 0:1]
    deginv = 1.0 / jnp.maximum(deg, 1.0)
    a = (agg_ref[0] + agg_ref[1]) * deginv
    t = (jnp.dot(a, wl_ref[...], preferred_element_type=jnp.float32)
         + b_ref[...]
         + jnp.dot(x_ref[...], wr_ref[...], preferred_element_type=jnp.float32))
    if final:
        ss = jnp.sum(t * t, axis=1, keepdims=True)
        t = t / jnp.maximum(jnp.sqrt(ss), 1e-12)
    else:
        t = jnp.maximum(t, 0.0)
    o_ref[...] = t


def _make_tc(final):
    return pl.pallas_call(
        functools.partial(_tc_body, final),
        grid=(GRID,),
        in_specs=[
            pl.BlockSpec((2, BN, D), lambda i: (0, i, 0)),
            pl.BlockSpec((2, BN, D), lambda i: (0, i, 0)),
            pl.BlockSpec((BN, D), lambda i: (i, 0)),
            pl.BlockSpec((D, D), lambda i: (0, 0)),
            pl.BlockSpec((1, D), lambda i: (0, 0)),
            pl.BlockSpec((D, D), lambda i: (0, 0)),
        ],
        out_specs=pl.BlockSpec((BN, D), lambda i: (i, 0)),
        out_shape=jax.ShapeDtypeStruct((NP, D), jnp.float32),
    )


_tc_layer1 = _make_tc(False)
_tc_layer2 = _make_tc(True)


def kernel(x, edge_index, W1l, b1l, W1r, W2l, b2l, W2r):
    src = edge_index[0].astype(jnp.int32)
    dst = edge_index[1].astype(jnp.int32)
    srcp = jnp.concatenate([src, jnp.zeros((EP - E,), jnp.int32)])
    dstp = jnp.concatenate([dst, jnp.full((EP - E,), N, jnp.int32)])
    xp = jnp.pad(x, ((0, NP - N), (0, 0)))
    zrows = jnp.zeros((C, D), jnp.float32)
    orows = jnp.ones((C, D), jnp.float32)

    degs = _sc_deg(orows, srcp, dstp, zrows)[0].reshape(2, NP, D)
    agg1 = _sc_agg(xp, srcp, dstp, zrows)[0].reshape(2, NP, D)
    h = _tc_layer1(agg1, degs, xp, W1l.T, b1l.reshape(1, D), W1r.T)

    agg2 = _sc_agg(h, srcp, dstp, zrows)[0].reshape(2, NP, D)
    out = _tc_layer2(agg2, degs, h, W2l.T, b2l.reshape(1, D), W2r.T)
    return out[:N]


# speedup vs baseline: 3.9835x; 1.2815x over previous
import functools

import jax
import jax.numpy as jnp
from jax import lax
from jax.experimental import pallas as pl
from jax.experimental.pallas import tpu as pltpu
from jax.experimental.pallas import tpu_sc as plsc

N = 10000
E = 320000
D = 128
DW = 144

NP = 10240
NW = 32
EP = 327680
EW = EP // NW
C = 128
CHUNKS = EW // C
CH = CHUNKS // 2
RPT = NP // 16
DEGW = 16

_MESH = plsc.VectorSubcoreMesh(core_axis_name="c", subcore_axis_name="s")


def _sc_body(with_gather, table, srcs3, dsts3, zrows, agg_out, src_t, dst_t,
             buf0, buf1, acc, g0, g1, s0, s1):
    c = lax.axis_index("c")
    s = lax.axis_index("s")
    wid = s * 2 + c
    rbase = s * RPT
    obase = c * NP + rbase

    pltpu.sync_copy(zrows, buf0)
    pltpu.sync_copy(zrows, buf1)
    for jj in range(RPT // C):
        pltpu.sync_copy(buf0, acc.at[pl.ds(rbase + jj * C, C)])
    if not with_gather:
        pltpu.sync_copy(table.at[pl.ds(0, C)], buf0)
    plsc.subcore_barrier()

    def _wait_scatter(buf, sem):
        pltpu.make_async_copy(buf, acc.at[dst_t.at[0]], sem).wait()

    def _wait_gather(buf, sem):
        pltpu.make_async_copy(table.at[src_t.at[0]], buf, sem).wait()

    def _gather(j, buf, sem):
        pltpu.async_copy(table.at[src_t.at[j]], buf, sem)

    def _scatter(j, buf, sem):
        pltpu.async_copy(buf, acc.at[dst_t.at[j]], sem, add=True)

    for h in range(CHUNKS // CH):
        pltpu.sync_copy(dsts3.at[wid, pl.ds(h * CH, CH)], dst_t)
        if with_gather:
            pltpu.sync_copy(srcs3.at[wid, pl.ds(h * CH, CH)], src_t)
            _gather(0, buf0, g0)
            _gather(1, buf1, g1)

            def step(m, carry):
                j0 = 2 * m
                _wait_gather(buf0, g0)
                _scatter(j0, buf0, s0)
                _wait_scatter(buf0, s0)
                _gather(j0 + 2, buf0, g0)
                _wait_gather(buf1, g1)
                _scatter(j0 + 1, buf1, s1)
                _wait_scatter(buf1, s1)
                _gather(j0 + 3, buf1, g1)
                return carry

            lax.fori_loop(0, CH // 2 - 1, step, 0)
            _wait_gather(buf0, g0)
            _scatter(CH - 2, buf0, s0)
            _wait_scatter(buf0, s0)
            _wait_gather(buf1, g1)
            _scatter(CH - 1, buf1, s1)
            _wait_scatter(buf1, s1)
        else:
            def dstep(m, carry):
                for b in range(8):
                    _scatter(m * 8 + b, buf0, s0)
                for b in range(8):
                    _wait_scatter(buf0, s0)
                return carry

            lax.fori_loop(0, CH // 8, dstep, 0)

    plsc.subcore_barrier()

    for jj in range(RPT // C):
        pltpu.sync_copy(acc.at[pl.ds(rbase + jj * C, C)], buf0)
        pltpu.sync_copy(buf0, agg_out.at[pl.ds(obase + jj * C, C)])


def _make_sc(with_gather):
    return pl.kernel(
        functools.partial(_sc_body, with_gather),
        out_type=[jax.ShapeDtypeStruct((2 * NP, D), jnp.float32)],
        mesh=_MESH,
        scratch_types=[
            pltpu.VMEM((CH, C), jnp.int32),
            pltpu.VMEM((CH, C), jnp.int32),
            pltpu.VMEM((C, D), jnp.float32),
            pltpu.VMEM((C, D), jnp.float32),
            pltpu.VMEM_SHARED((NP, D), jnp.float32),
            pltpu.SemaphoreType.DMA,
            pltpu.SemaphoreType.DMA,
            pltpu.SemaphoreType.DMA,
            pltpu.SemaphoreType.DMA,
        ],
    )


_sc_agg = _make_sc(True)
_sc_deg = _make_sc(False)

BN = 512
GRID = NP // BN


def _tc_body(final, agg_ref, degs_ref, x_ref, wl_ref, b_ref, wr_ref, o_ref):
    deg = degs_ref[0, :, 0:1] + degs_ref[1, :, 0:1]
    deginv = 1.0 / jnp.maximum(deg, 1.0)
    a = (agg_ref[0] + agg_ref[1]) * deginv
    t = (jnp.dot(a, wl_ref[...], preferred_element_type=jnp.float32)
         + b_ref[...]
         + jnp.dot(x_ref[...], wr_ref[...], preferred_element_type=jnp.float32))
    if final:
        ss = jnp.sum(t * t, axis=1, keepdims=True)
        t = t / jnp.maximum(jnp.sqrt(ss), 1e-12)
    else:
        t = jnp.maximum(t, 0.0)
    o_ref[...] = t


def _make_tc(final):
    return pl.pallas_call(
        functools.partial(_tc_body, final),
        grid=(GRID,),
        in_specs=[
            pl.BlockSpec((2, BN, D), lambda i: (0, i, 0)),
            pl.BlockSpec((2, BN, D), lambda i: (0, i, 0)),
            pl.BlockSpec((BN, D), lambda i: (i, 0)),
            pl.BlockSpec((D, D), lambda i: (0, 0)),
            pl.BlockSpec((1, D), lambda i: (0, 0)),
            pl.BlockSpec((D, D), lambda i: (0, 0)),
        ],
        out_specs=pl.BlockSpec((BN, D), lambda i: (i, 0)),
        out_shape=jax.ShapeDtypeStruct((NP, D), jnp.float32),
    )


_tc_layer1 = _make_tc(False)
_tc_layer2 = _make_tc(True)


def kernel(x, edge_index, W1l, b1l, W1r, W2l, b2l, W2r):
    src = edge_index[0].astype(jnp.int32)
    dst = edge_index[1].astype(jnp.int32)
    srcp = jnp.concatenate([src, jnp.zeros((EP - E,), jnp.int32)])
    dstp = jnp.concatenate([dst, jnp.full((EP - E,), N, jnp.int32)])
    xp = jnp.pad(x, ((0, NP - N), (0, 0)))
    zrows = jnp.zeros((C, D), jnp.float32)
    orows = jnp.ones((C, D), jnp.float32)

    srcp = srcp.reshape(NW, CHUNKS, C)
    dstp = dstp.reshape(NW, CHUNKS, C)
    degs = _sc_deg(orows, srcp, dstp, zrows)[0].reshape(2, NP, D)
    agg1 = _sc_agg(xp, srcp, dstp, zrows)[0].reshape(2, NP, D)
    h = _tc_layer1(agg1, degs, xp, W1l.T, b1l.reshape(1, D), W1r.T)

    agg2 = _sc_agg(h, srcp, dstp, zrows)[0].reshape(2, NP, D)
    out = _tc_layer2(agg2, degs, h, W2l.T, b2l.reshape(1, D), W2r.T)
    return out[:N]
